# Initial kernel scaffold; baseline (speedup 1.0000x reference)
import jax
import jax.numpy as jnp
from jax.experimental import pallas as pl


def kernel(mh_output, W, b):
    raise NotImplementedError("write your pallas kernel here")


# speedup vs baseline: 4.3962x; 4.3962x over previous
import functools

import jax
import jax.numpy as jnp
from jax import lax
from jax.experimental import pallas as pl

N_EMBED = 4096
NUM_EXPERTS = 64
TOP_K = 8
TOKENS = 4 * 4096
BLOCK_T = 512

_NEG_INF = float("-inf")


def _router_block_kernel(x_ref, w_ref, b_ref, out_ref, idx_ref):
    x = x_ref[...]
    w = w_ref[...]
    logits = jax.lax.dot_general(
        x, w,
        dimension_numbers=(((1,), (1,)), ((), ())),
        preferred_element_type=jnp.float32,
    ) + b_ref[...]

    e_iota = lax.broadcasted_iota(jnp.int32, (BLOCK_T, NUM_EXPERTS), 1)
    work = logits
    vals = []
    idxs = []
    for _ in range(TOP_K):
        m = jnp.max(work, axis=1, keepdims=True)
        amax = jnp.min(
            jnp.where(work == m, e_iota, NUM_EXPERTS), axis=1, keepdims=True
        )
        vals.append(m)
        idxs.append(amax)
        work = jnp.where(e_iota == amax, _NEG_INF, work)

    top_vals = jnp.concatenate(vals, axis=1)
    top_idxs = jnp.concatenate(idxs, axis=1)

    m0 = top_vals[:, 0:1]
    denom = jnp.sum(jnp.exp(top_vals - m0), axis=1, keepdims=True)
    selected = work == _NEG_INF
    probs = jnp.where(selected, jnp.exp(logits - m0) / denom, 0.0)

    out_ref[...] = probs
    idx_ref[...] = top_idxs


@functools.partial(jax.jit, static_argnames=())
def _run(x2d, W, b2d):
    grid = (TOKENS // BLOCK_T,)
    out, idx = pl.pallas_call(
        _router_block_kernel,
        grid=grid,
        in_specs=[
            pl.BlockSpec((BLOCK_T, N_EMBED), lambda i: (i, 0)),
            pl.BlockSpec((NUM_EXPERTS, N_EMBED), lambda i: (0, 0)),
            pl.BlockSpec((1, NUM_EXPERTS), lambda i: (0, 0)),
        ],
        out_specs=[
            pl.BlockSpec((BLOCK_T, NUM_EXPERTS), lambda i: (i, 0)),
            pl.BlockSpec((BLOCK_T, TOP_K), lambda i: (i, 0)),
        ],
        out_shape=[
            jax.ShapeDtypeStruct((TOKENS, NUM_EXPERTS), jnp.float32),
            jax.ShapeDtypeStruct((TOKENS, TOP_K), jnp.int32),
        ],
    )(x2d, W, b2d)
    return out, idx


def kernel(mh_output, W, b):
    B, S, E = mh_output.shape
    x2d = mh_output.reshape(B * S, E)
    out, idx = _run(x2d, W, b.reshape(1, NUM_EXPERTS))
    return out.reshape(B, S, NUM_EXPERTS), idx.reshape(B, S, TOP_K)
